# baseline (speedup 1.0000x reference)
import jax
import jax.numpy as jnp
from jax import lax
from jax.experimental import pallas as pl
from jax.experimental.pallas import tpu as pltpu

_WAY = 5
_SHOT = 5
_K = 3
_QB = 5


def _top3sum(x):
    m1 = jnp.max(x, axis=1, keepdims=True)
    eq1 = x == m1
    c1 = jnp.sum(eq1.astype(jnp.float32), axis=1, keepdims=True)
    x2 = jnp.where(eq1, -jnp.inf, x)
    m2 = jnp.max(x2, axis=1, keepdims=True)
    eq2 = x2 == m2
    c2 = jnp.sum(eq2.astype(jnp.float32), axis=1, keepdims=True)
    x3 = jnp.where(eq2, -jnp.inf, x2)
    m3 = jnp.max(x3, axis=1, keepdims=True)
    k1 = jnp.minimum(c1, 3.0)
    k2 = jnp.minimum(c2, 3.0 - k1)
    k3 = jnp.maximum(3.0 - k1 - k2, 0.0)
    t = m1 * k1
    t = t + jnp.where(k2 > 0, m2 * k2, 0.0)
    t = t + jnp.where(k3 > 0, m3 * k3, 0.0)
    return t


def _consume_triple(t1, t2, t3):
    C = t1
    D = jnp.zeros_like(t1)
    total = jnp.zeros((t1.shape[0], 1), jnp.float32)
    k = jnp.zeros((t1.shape[0], 1), jnp.float32)
    for i in range(_K):
        mx = jnp.max(C, axis=1, keepdims=True)
        eq = C == mx
        cnt = jnp.sum(eq.astype(jnp.float32), axis=1, keepdims=True)
        take = jnp.minimum(cnt, 3.0 - k)
        total = total + jnp.where(take > 0, mx * take, 0.0)
        k = k + take
        if i < _K - 1:
            nxt = jnp.where(D == 0, t2, t3)
            C = jnp.where(eq, nxt, C)
            D = D + eq.astype(jnp.float32)
    return total


def _merge3(A, B):
    a1, a2, a3 = A
    b1, b2, b3 = B
    t1 = jnp.maximum(a1, b1)
    x1 = jnp.minimum(a1, b1)
    t2 = jnp.maximum(a2, x1)
    y = jnp.minimum(a2, x1)
    t3 = jnp.maximum(a3, y)
    t2b = jnp.maximum(t2, b2)
    z = jnp.minimum(t2, b2)
    t3b = jnp.maximum(t3, z)
    t3c = jnp.maximum(t3b, b3)
    return t1, t2b, t3c


def _top3sum_chunked(x):
    R, N = x.shape
    CH = 128
    nch = -(-N // CH)
    pad = nch * CH - N
    if pad:
        x = jnp.concatenate(
            [x, jnp.full((R, pad), -jnp.inf, jnp.float32)], axis=1)
    chunks = [x[:, i * CH:(i + 1) * CH] for i in range(nch)]
    s2 = []
    for i in range(0, nch - 1, 2):
        a, b = chunks[i], chunks[i + 1]
        s2.append((jnp.maximum(a, b), jnp.minimum(a, b)))
    if nch % 2:
        neg = jnp.full((R, CH), -jnp.inf, jnp.float32)
        s2.append((chunks[-1], neg))
    s3 = []
    for i in range(0, len(s2) - 1, 2):
        (d0, e0), (d1, e1) = s2[i], s2[i + 1]
        h = jnp.maximum(d0, d1)
        l = jnp.minimum(d0, d1)
        m = jnp.maximum(e0, e1)
        p = jnp.maximum(l, m)
        q = jnp.minimum(l, m)
        s3.append((h, p, q))
    if len(s2) % 2:
        d0, e0 = s2[-1]
        neg = jnp.full((R, CH), -jnp.inf, jnp.float32)
        s3.append((d0, e0, neg))
    while len(s3) > 1:
        nxt = [_merge3(s3[i], s3[i + 1]) for i in range(0, len(s3) - 1, 2)]
        if len(s3) % 2:
            nxt.append(s3[-1])
        s3 = nxt
    return _consume_triple(*s3[0])


def _body(q_ref, s_ref, wsw_ref, par_ref, out_ref):
    C, U = s_ref.shape
    HW = q_ref.shape[1]
    SHW = _SHOT * HW
    ws_b = par_ref[0, 0]
    msn_w = par_ref[0, 1]
    msn_b = par_ref[0, 2]
    wm_w = par_ref[0, 3]
    wm_b = par_ref[0, 4]

    s = s_ref[...]
    sn = s * lax.rsqrt(jnp.sum(s * s, axis=0, keepdims=True))
    pw = jnp.zeros((C, 1), jnp.float32)
    for v in range(_WAY):
        pv = jnp.mean(sn[:, v * SHW:(v + 1) * SHW], axis=1, keepdims=True)
        pw = pw + wsw_ref[0, v] * pv
    ssw = jax.nn.sigmoid(jnp.sum(pw * sn, axis=0, keepdims=True) + ws_b)

    R = _QB * HW
    q = q_ref[...].reshape(R, C)
    qn = q * lax.rsqrt(jnp.sum(q * q, axis=1, keepdims=True))

    outs = []
    for v in range(_WAY):
        Av = jnp.dot(qn, sn[:, v * SHW:(v + 1) * SHW],
                     preferred_element_type=jnp.float32)
        M = Av[:, 0:HW]
        for b in range(1, _SHOT):
            M = jnp.maximum(M, Av[:, b * HW:(b + 1) * HW])
        rel = _top3sum_chunked(msn_w * M + msn_b)
        w = jax.nn.sigmoid(wm_w * rel + wm_b)
        r2 = _top3sum_chunked(Av * ssw[:, v * SHW:(v + 1) * SHW])
        pq = jnp.sum((w * r2).reshape(_QB, HW), axis=1)
        outs.append(pq.reshape(_QB, 1, 1))
    out_ref[...] = jnp.concatenate(outs, axis=2)


@jax.jit
def kernel(x1, x2, ws_w, ws_b, msn_w, msn_b, wm_w, wm_b):
    Q, C, H, W = x1.shape
    HW = H * W
    S = x2.shape[0]
    U = S * HW

    x1t = x1.reshape(Q, C, HW).transpose(0, 2, 1)
    s_cols = x2.reshape(S, C, HW).transpose(1, 0, 2).reshape(C, U)
    s_perm = (s_cols.reshape(C, _WAY, HW, _SHOT)
              .transpose(0, 1, 3, 2).reshape(C, U))
    wsw = jnp.asarray(ws_w, jnp.float32).reshape(1, _WAY)
    par = jnp.stack([ws_b, msn_w, msn_b, wm_w, wm_b]).astype(
        jnp.float32).reshape(1, 5)

    out = pl.pallas_call(
        _body,
        grid=(Q // _QB,),
        in_specs=[
            pl.BlockSpec((_QB, HW, C), lambda i: (i, 0, 0)),
            pl.BlockSpec((C, U), lambda i: (0, 0)),
            pl.BlockSpec((1, _WAY), lambda i: (0, 0)),
            pl.BlockSpec((1, 5), lambda i: (0, 0)),
        ],
        out_specs=pl.BlockSpec((_QB, 1, _WAY), lambda i: (i, 0, 0)),
        out_shape=jax.ShapeDtypeStruct((Q, 1, _WAY), jnp.float32),
        compiler_params=pltpu.CompilerParams(
            dimension_semantics=("parallel",)),
    )(x1t, s_perm, wsw, par)
    return out.reshape(Q, _WAY)


# speedup vs baseline: 1.0422x; 1.0422x over previous
import jax
import jax.numpy as jnp
from jax import lax
from jax.experimental import pallas as pl
from jax.experimental.pallas import tpu as pltpu

_WAY = 5
_SHOT = 5
_K = 3
_QB = 5


def _top3sum(x):
    m1 = jnp.max(x, axis=1, keepdims=True)
    eq1 = x == m1
    c1 = jnp.sum(eq1.astype(jnp.float32), axis=1, keepdims=True)
    x2 = jnp.where(eq1, -jnp.inf, x)
    m2 = jnp.max(x2, axis=1, keepdims=True)
    eq2 = x2 == m2
    c2 = jnp.sum(eq2.astype(jnp.float32), axis=1, keepdims=True)
    x3 = jnp.where(eq2, -jnp.inf, x2)
    m3 = jnp.max(x3, axis=1, keepdims=True)
    k1 = jnp.minimum(c1, 3.0)
    k2 = jnp.minimum(c2, 3.0 - k1)
    k3 = jnp.maximum(3.0 - k1 - k2, 0.0)
    t = m1 * k1
    t = t + jnp.where(k2 > 0, m2 * k2, 0.0)
    t = t + jnp.where(k3 > 0, m3 * k3, 0.0)
    return t


def _consume_triple(t1, t2, t3):
    C = t1
    D = jnp.zeros_like(t1)
    total = jnp.zeros((t1.shape[0], 1), jnp.float32)
    k = jnp.zeros((t1.shape[0], 1), jnp.float32)
    for i in range(_K):
        mx = jnp.max(C, axis=1, keepdims=True)
        eq = C == mx
        cnt = jnp.sum(eq.astype(jnp.float32), axis=1, keepdims=True)
        take = jnp.minimum(cnt, 3.0 - k)
        total = total + jnp.where(take > 0, mx * take, 0.0)
        k = k + take
        if i < _K - 1:
            nxt = jnp.where(D == 0, t2, t3)
            C = jnp.where(eq, nxt, C)
            D = D + eq.astype(jnp.float32)
    return total


def _merge3(A, B):
    a1, a2, a3 = A
    b1, b2, b3 = B
    t1 = jnp.maximum(a1, b1)
    x1 = jnp.minimum(a1, b1)
    t2 = jnp.maximum(a2, x1)
    y = jnp.minimum(a2, x1)
    t3 = jnp.maximum(a3, y)
    t2b = jnp.maximum(t2, b2)
    z = jnp.minimum(t2, b2)
    t3b = jnp.maximum(t3, z)
    t3c = jnp.maximum(t3b, b3)
    return t1, t2b, t3c


def _top3sum_chunked(x):
    R, N = x.shape
    CH = 128
    nch = -(-N // CH)
    pad = nch * CH - N
    if pad:
        x = jnp.concatenate(
            [x, jnp.full((R, pad), -jnp.inf, jnp.float32)], axis=1)
    chunks = [x[:, i * CH:(i + 1) * CH] for i in range(nch)]
    s2 = []
    for i in range(0, nch - 1, 2):
        a, b = chunks[i], chunks[i + 1]
        s2.append((jnp.maximum(a, b), jnp.minimum(a, b)))
    if nch % 2:
        neg = jnp.full((R, CH), -jnp.inf, jnp.float32)
        s2.append((chunks[-1], neg))
    s3 = []
    for i in range(0, len(s2) - 1, 2):
        (d0, e0), (d1, e1) = s2[i], s2[i + 1]
        h = jnp.maximum(d0, d1)
        l = jnp.minimum(d0, d1)
        m = jnp.maximum(e0, e1)
        p = jnp.maximum(l, m)
        q = jnp.minimum(l, m)
        s3.append((h, p, q))
    if len(s2) % 2:
        d0, e0 = s2[-1]
        neg = jnp.full((R, CH), -jnp.inf, jnp.float32)
        s3.append((d0, e0, neg))
    while len(s3) > 1:
        nxt = [_merge3(s3[i], s3[i + 1]) for i in range(0, len(s3) - 1, 2)]
        if len(s3) % 2:
            nxt.append(s3[-1])
        s3 = nxt
    return _consume_triple(*s3[0])


def _prep_body(s_ref, wsw_ref, par_ref, sn_ref, ssw_ref):
    C, U = s_ref.shape
    SHW = U // _WAY
    s = s_ref[...]
    sn = s * lax.rsqrt(jnp.sum(s * s, axis=0, keepdims=True))
    sn_ref[...] = sn
    pw = jnp.zeros((C, 1), jnp.float32)
    for v in range(_WAY):
        pv = jnp.mean(sn[:, v * SHW:(v + 1) * SHW], axis=1, keepdims=True)
        pw = pw + wsw_ref[0, v] * pv
    ssw_ref[...] = jax.nn.sigmoid(
        jnp.sum(pw * sn, axis=0, keepdims=True) + par_ref[0, 0])


def _body(q_ref, sn_ref, ssw_ref, par_ref, e_ref, out_ref):
    C, U = sn_ref.shape
    HW = q_ref.shape[1]
    SHW = _SHOT * HW
    msn_w = par_ref[0, 1]
    msn_b = par_ref[0, 2]
    wm_w = par_ref[0, 3]
    wm_b = par_ref[0, 4]
    sn = sn_ref[...]
    ssw = ssw_ref[...]

    R = _QB * HW
    q = q_ref[...].reshape(R, C)
    qn = q * lax.rsqrt(jnp.sum(q * q, axis=1, keepdims=True))

    cols = []
    for v in range(_WAY):
        Av = jnp.dot(qn, sn[:, v * SHW:(v + 1) * SHW],
                     preferred_element_type=jnp.float32)
        M = Av[:, 0:HW]
        for b in range(1, _SHOT):
            M = jnp.maximum(M, Av[:, b * HW:(b + 1) * HW])
        rel = _top3sum(msn_w * M + msn_b)
        w = jax.nn.sigmoid(wm_w * rel + wm_b)
        r2 = _top3sum_chunked(Av * ssw[:, v * SHW:(v + 1) * SHW])
        cols.append(w * r2)
    WR = jnp.concatenate(cols, axis=1)
    out = jnp.dot(e_ref[...], WR, preferred_element_type=jnp.float32)
    out_ref[...] = out.reshape(_QB, 1, _WAY)


@jax.jit
def kernel(x1, x2, ws_w, ws_b, msn_w, msn_b, wm_w, wm_b):
    Q, C, H, W = x1.shape
    HW = H * W
    S = x2.shape[0]
    U = S * HW

    x1t = x1.reshape(Q, C, HW).transpose(0, 2, 1)
    s_cols = x2.reshape(S, C, HW).transpose(1, 0, 2).reshape(C, U)
    s_perm = (s_cols.reshape(C, _WAY, HW, _SHOT)
              .transpose(0, 1, 3, 2).reshape(C, U))
    wsw = jnp.asarray(ws_w, jnp.float32).reshape(1, _WAY)
    par = jnp.stack([ws_b, msn_w, msn_b, wm_w, wm_b]).astype(
        jnp.float32).reshape(1, 5)

    sn, ssw = pl.pallas_call(
        _prep_body,
        out_shape=[
            jax.ShapeDtypeStruct((C, U), jnp.float32),
            jax.ShapeDtypeStruct((1, U), jnp.float32),
        ],
    )(s_perm, wsw, par)

    e_sel = jnp.repeat(jnp.eye(_QB, dtype=jnp.float32), HW, axis=1)

    out = pl.pallas_call(
        _body,
        grid=(Q // _QB,),
        in_specs=[
            pl.BlockSpec((_QB, HW, C), lambda i: (i, 0, 0)),
            pl.BlockSpec((C, U), lambda i: (0, 0)),
            pl.BlockSpec((1, U), lambda i: (0, 0)),
            pl.BlockSpec((1, 5), lambda i: (0, 0)),
            pl.BlockSpec((_QB, _QB * HW), lambda i: (0, 0)),
        ],
        out_specs=pl.BlockSpec((_QB, 1, _WAY), lambda i: (i, 0, 0)),
        out_shape=jax.ShapeDtypeStruct((Q, 1, _WAY), jnp.float32),
        compiler_params=pltpu.CompilerParams(
            dimension_semantics=("parallel",)),
    )(x1t, sn, ssw, par, e_sel)
    return out.reshape(Q, _WAY)
